# baseline (speedup 1.0000x reference)
import functools

import jax
import jax.numpy as jnp
from jax import lax
from jax.experimental import pallas as pl
from jax.experimental.pallas import tpu as pltpu
from jax.experimental.pallas import tpu_sc as plsc

N = 10000
D = 128
E = 320000
H = 4
ROW = 144
SIM_T = 0.1
EPS = 1e-8

NC, NS = 2, 16
NW = NC * NS
EPT = E // NW
B = 80
NB = EPT // B
BR = 1000


def _tables_body(l_ref, r_ref, w_ref, al_ref, ar_ref):
    w = w_ref[:]
    for x_ref, o_ref in ((l_ref, al_ref), (r_ref, ar_ref)):
        x = x_ref[:]
        cols = []
        for h in range(H):
            wl = x * w[h:h + 1, :]
            s = jnp.sum(wl * wl, axis=1, keepdims=True)
            cols.append(1.0 / jnp.maximum(jnp.sqrt(s), EPS))
        pad = jnp.zeros((BR, ROW - D - H), jnp.float32)
        o_ref[:] = jnp.concatenate([x] + cols + [pad], axis=1)


def _build_tables(left, right, w2d):
    return pl.pallas_call(
        _tables_body,
        grid=(N // BR,),
        in_specs=[
            pl.BlockSpec((BR, D), lambda i: (i, 0)),
            pl.BlockSpec((BR, D), lambda i: (i, 0)),
            pl.BlockSpec((H, D), lambda i: (0, 0)),
        ],
        out_specs=[
            pl.BlockSpec((BR, ROW), lambda i: (i, 0)),
            pl.BlockSpec((BR, ROW), lambda i: (i, 0)),
        ],
        out_shape=[
            jax.ShapeDtypeStruct((N, ROW), jnp.float32),
            jax.ShapeDtypeStruct((N, ROW), jnp.float32),
        ],
    )(left, right, w2d)


def _edge_sim(al, ar, edge_index, w2d):
    mesh = plsc.VectorSubcoreMesh(core_axis_name="c", subcore_axis_name="s")

    @functools.partial(
        pl.kernel,
        out_type=jax.ShapeDtypeStruct((E,), jnp.float32),
        mesh=mesh,
        compiler_params=pltpu.CompilerParams(needs_layout_passes=False,
                                             use_tc_tiling_on_sc=False),
        scratch_types=[
            pltpu.VMEM((EPT,), jnp.int32),
            pltpu.VMEM((EPT,), jnp.int32),
            pltpu.VMEM((3, B, ROW), jnp.float32),
            pltpu.VMEM((3, B, ROW), jnp.float32),
            pltpu.VMEM((EPT,), jnp.float32),
            pltpu.VMEM((H, D), jnp.float32),
            pltpu.SemaphoreType.DMA((3,)),
        ],
    )
    def run(al_hbm, ar_hbm, ei_hbm, w_hbm, out_hbm,
            src_v, dst_v, lbuf, rbuf, out_v, w_v, sems):
        wid = lax.axis_index("s") * NC + lax.axis_index("c")
        base = wid * EPT
        pltpu.sync_copy(ei_hbm.at[pl.ds(base, EPT)], src_v)
        pltpu.sync_copy(ei_hbm.at[pl.ds(E + base, EPT)], dst_v)
        pltpu.sync_copy(w_hbm, w_v)
        w2 = []
        for h in range(H):
            row = []
            for c in range(8):
                wv = w_v[h, pl.ds(c * 16, 16)]
                row.append(wv * wv)
            w2.append(row)

        def start(block, lb, rb, sem):
            off = block * B
            pltpu.async_copy(al_hbm.at[src_v.at[pl.ds(off, B)]], lb, sem)
            pltpu.async_copy(ar_hbm.at[dst_v.at[pl.ds(off, B)]], rb, sem)

        def wait(lb, rb, sem):
            pltpu.make_async_copy(al_hbm.at[src_v.at[pl.ds(0, B)]], lb, sem).wait()
            pltpu.make_async_copy(ar_hbm.at[dst_v.at[pl.ds(0, B)]], rb, sem).wait()

        lane = lax.iota(jnp.int32, 16)

        def compute(block, lb, rb):
            def group(g, carry):
                sims = jnp.zeros((16,), jnp.float32)
                for j in range(16):
                    e = g * 16 + j
                    lcs = [lb[e, pl.ds(c * 16, 16)] for c in range(8)]
                    rcs = [rb[e, pl.ds(c * 16, 16)] for c in range(8)]
                    qs = [lcs[c] * rcs[c] for c in range(8)]
                    lv = lb[e, pl.ds(D, 16)]
                    rv = rb[e, pl.ds(D, 16)]
                    tot = None
                    for h in range(H):
                        acc = qs[0] * w2[h][0]
                        for c in range(1, 8):
                            acc = acc + qs[c] * w2[h][c]
                        term = acc * (lv[h] * rv[h])
                        tot = term if tot is None else tot + term
                    sim = jnp.sum(tot) * jnp.float32(1.0 / H)
                    sims = jnp.where(lane == j, sim, sims)
                sims = jnp.where(sims < SIM_T, jnp.float32(0.0), sims)
                out_v[pl.ds(block * B + g * 16, 16)] = sims
                return carry
            lax.fori_loop(0, B // 16, group, 0)

        slots = tuple((lbuf.at[j], rbuf.at[j], sems.at[j]) for j in range(3))
        for j in range(3):
            start(j, *slots[j])

        def outer(kk, carry):
            for j in range(3):
                b = 3 * kk + j
                lb, rb, sem = slots[j]
                wait(lb, rb, sem)
                compute(b, lb, rb)

                @pl.when(b + 3 < NB)
                def _():
                    start(b + 3, lb, rb, sem)
            return carry

        lax.fori_loop(0, NB // 3, outer, 0)
        for j in range(NB - 3 * (NB // 3)):
            b = 3 * (NB // 3) + j
            lb, rb, sem = slots[j]
            wait(lb, rb, sem)
            compute(b, lb, rb)
        pltpu.sync_copy(out_v, out_hbm.at[pl.ds(base, EPT)])

    return run(al, ar, edge_index, w2d)


def kernel(left_features, right_features, edge_index, W):
    w2d = W.reshape(H, D)
    al, ar = _build_tables(left_features, right_features, w2d)
    return _edge_sim(al, ar, edge_index.reshape(2 * E), w2d)


# speedup vs baseline: 1.1374x; 1.1374x over previous
import functools

import jax
import jax.numpy as jnp
from jax import lax
from jax.experimental import pallas as pl
from jax.experimental.pallas import tpu as pltpu
from jax.experimental.pallas import tpu_sc as plsc

N = 10000
D = 128
E = 320000
H = 4
ROW = 144
SIM_T = 0.1
EPS = 1e-8

NC, NS = 2, 16
NW = NC * NS
EPT = E // NW
B = 80
NB = EPT // B
BR = 1000


def _tables_body(l_ref, r_ref, w_ref, al_ref, ar_ref):
    w = w_ref[:]
    for x_ref, o_ref in ((l_ref, al_ref), (r_ref, ar_ref)):
        x = x_ref[:]
        cols = []
        for h in range(H):
            wl = x * w[h:h + 1, :]
            s = jnp.sum(wl * wl, axis=1, keepdims=True)
            cols.append(1.0 / jnp.maximum(jnp.sqrt(s), EPS))
        pad = jnp.zeros((BR, ROW - D - H), jnp.float32)
        o_ref[:] = jnp.concatenate([x] + cols + [pad], axis=1)


def _build_tables(left, right, w2d):
    return pl.pallas_call(
        _tables_body,
        grid=(N // BR,),
        in_specs=[
            pl.BlockSpec((BR, D), lambda i: (i, 0)),
            pl.BlockSpec((BR, D), lambda i: (i, 0)),
            pl.BlockSpec((H, D), lambda i: (0, 0)),
        ],
        out_specs=[
            pl.BlockSpec((BR, ROW), lambda i: (i, 0)),
            pl.BlockSpec((BR, ROW), lambda i: (i, 0)),
        ],
        out_shape=[
            jax.ShapeDtypeStruct((N, ROW), jnp.float32),
            jax.ShapeDtypeStruct((N, ROW), jnp.float32),
        ],
    )(left, right, w2d)


def _edge_sim(al, ar, edge_index, w2d):
    mesh = plsc.VectorSubcoreMesh(core_axis_name="c", subcore_axis_name="s")

    @functools.partial(
        pl.kernel,
        out_type=jax.ShapeDtypeStruct((E,), jnp.float32),
        mesh=mesh,
        compiler_params=pltpu.CompilerParams(needs_layout_passes=False,
                                             use_tc_tiling_on_sc=False),
        scratch_types=[
            pltpu.VMEM((EPT,), jnp.int32),
            pltpu.VMEM((EPT,), jnp.int32),
            pltpu.VMEM((2, B, ROW), jnp.float32),
            pltpu.VMEM((2, B, ROW), jnp.float32),
            pltpu.VMEM((EPT,), jnp.float32),
            pltpu.VMEM((H, D), jnp.float32),
            pltpu.SemaphoreType.DMA((2,)),
        ],
    )
    def run(al_hbm, ar_hbm, ei_hbm, w_hbm, out_hbm,
            src_v, dst_v, lbuf, rbuf, out_v, w_v, sems):
        wid = lax.axis_index("s") * NC + lax.axis_index("c")
        base = wid * EPT
        pltpu.sync_copy(ei_hbm.at[pl.ds(base, EPT)], src_v)
        pltpu.sync_copy(ei_hbm.at[pl.ds(E + base, EPT)], dst_v)
        pltpu.sync_copy(w_hbm, w_v)
        w2 = []
        for h in range(H):
            row = []
            for c in range(8):
                wv = w_v[h, pl.ds(c * 16, 16)]
                row.append(wv * wv)
            w2.append(row)

        def start(block, lb, rb, sem):
            off = block * B
            pltpu.async_copy(al_hbm.at[src_v.at[pl.ds(off, B)]], lb, sem)
            pltpu.async_copy(ar_hbm.at[dst_v.at[pl.ds(off, B)]], rb, sem)

        def wait(lb, rb, sem):
            pltpu.make_async_copy(al_hbm.at[src_v.at[pl.ds(0, B)]], lb, sem).wait()
            pltpu.make_async_copy(ar_hbm.at[dst_v.at[pl.ds(0, B)]], rb, sem).wait()

        lane = lax.iota(jnp.int32, 16)

        def compute(block, lb, rb):
            def group(g, carry):
                sims = jnp.zeros((16,), jnp.float32)
                for j in range(16):
                    e = g * 16 + j
                    lcs = [lb[e, pl.ds(c * 16, 16)] for c in range(8)]
                    rcs = [rb[e, pl.ds(c * 16, 16)] for c in range(8)]
                    qs = [lcs[c] * rcs[c] for c in range(8)]
                    lv = lb[e, pl.ds(D, 16)]
                    rv = rb[e, pl.ds(D, 16)]
                    tot = None
                    for h in range(H):
                        acc = qs[0] * w2[h][0]
                        for c in range(1, 8):
                            acc = acc + qs[c] * w2[h][c]
                        term = acc * (lv[h] * rv[h])
                        tot = term if tot is None else tot + term
                    sim = jnp.sum(tot) * jnp.float32(1.0 / H)
                    sims = jnp.where(lane == j, sim, sims)
                sims = jnp.where(sims < SIM_T, jnp.float32(0.0), sims)
                out_v[pl.ds(block * B + g * 16, 16)] = sims
                return carry
            lax.fori_loop(0, B // 16, group, 0)

        slots = tuple((lbuf.at[j], rbuf.at[j], sems.at[j]) for j in range(2))
        start(0, *slots[0])
        start(1, *slots[1])

        def outer(kk, carry):
            for j in range(2):
                b = 2 * kk + j
                lb, rb, sem = slots[j]
                wait(lb, rb, sem)
                compute(b, lb, rb)

                @pl.when(b + 2 < NB)
                def _():
                    start(b + 2, lb, rb, sem)
            return carry

        lax.fori_loop(0, NB // 2, outer, 0)
        for j in range(NB - 2 * (NB // 2)):
            b = 2 * (NB // 2) + j
            lb, rb, sem = slots[j]
            wait(lb, rb, sem)
            compute(b, lb, rb)
        pltpu.sync_copy(out_v, out_hbm.at[pl.ds(base, EPT)])

    return run(al, ar, edge_index, w2d)


def kernel(left_features, right_features, edge_index, W):
    w2d = W.reshape(H, D)
    al, ar = _build_tables(left_features, right_features, w2d)
    return _edge_sim(al, ar, edge_index.reshape(2 * E), w2d)


# speedup vs baseline: 1.1806x; 1.0380x over previous
import functools

import jax
import jax.numpy as jnp
from jax import lax
from jax.experimental import pallas as pl
from jax.experimental.pallas import tpu as pltpu
from jax.experimental.pallas import tpu_sc as plsc

N = 10000
D = 128
E = 320000
H = 4
NR_ = 16
SIM_T = 0.1
EPS = 1e-8

NC, NS = 2, 16
NW = NC * NS
EPT = E // NW
B = 80
NB = EPT // B
BR = 1000


def _norms_body(l_ref, r_ref, w_ref, nl_ref, nr_ref):
    w = w_ref[:]
    for x_ref, o_ref in ((l_ref, nl_ref), (r_ref, nr_ref)):
        x = x_ref[:]
        cols = []
        for h in range(H):
            wl = x * w[h:h + 1, :]
            s = jnp.sum(wl * wl, axis=1, keepdims=True)
            cols.append(1.0 / jnp.maximum(jnp.sqrt(s), EPS))
        pad = jnp.zeros((BR, NR_ - H), jnp.float32)
        o_ref[:] = jnp.concatenate(cols + [pad], axis=1)


def _build_norms(left, right, w2d):
    return pl.pallas_call(
        _norms_body,
        grid=(N // BR,),
        in_specs=[
            pl.BlockSpec((BR, D), lambda i: (i, 0)),
            pl.BlockSpec((BR, D), lambda i: (i, 0)),
            pl.BlockSpec((H, D), lambda i: (0, 0)),
        ],
        out_specs=[
            pl.BlockSpec((BR, NR_), lambda i: (i, 0)),
            pl.BlockSpec((BR, NR_), lambda i: (i, 0)),
        ],
        out_shape=[
            jax.ShapeDtypeStruct((N, NR_), jnp.float32),
            jax.ShapeDtypeStruct((N, NR_), jnp.float32),
        ],
    )(left, right, w2d)


def _edge_sim(lf, rf, nl, nr, edge_index, w2d):
    mesh = plsc.VectorSubcoreMesh(core_axis_name="c", subcore_axis_name="s")

    @functools.partial(
        pl.kernel,
        out_type=jax.ShapeDtypeStruct((E,), jnp.float32),
        mesh=mesh,
        compiler_params=pltpu.CompilerParams(needs_layout_passes=False,
                                             use_tc_tiling_on_sc=False),
        scratch_types=[
            pltpu.VMEM((EPT,), jnp.int32),
            pltpu.VMEM((EPT,), jnp.int32),
            pltpu.VMEM((2, B, D), jnp.float32),
            pltpu.VMEM((2, B, D), jnp.float32),
            pltpu.VMEM((2, B, NR_), jnp.float32),
            pltpu.VMEM((2, B, NR_), jnp.float32),
            pltpu.VMEM((EPT,), jnp.float32),
            pltpu.VMEM((H, D), jnp.float32),
            pltpu.SemaphoreType.DMA((2,)),
        ],
    )
    def run(lf_hbm, rf_hbm, nl_hbm, nr_hbm, ei_hbm, w_hbm, out_hbm,
            src_v, dst_v, lbuf, rbuf, nlbuf, nrbuf, out_v, w_v, sems):
        wid = lax.axis_index("s") * NC + lax.axis_index("c")
        base = wid * EPT
        pltpu.sync_copy(ei_hbm.at[pl.ds(base, EPT)], src_v)
        pltpu.sync_copy(ei_hbm.at[pl.ds(E + base, EPT)], dst_v)
        pltpu.sync_copy(w_hbm, w_v)
        w2 = []
        for h in range(H):
            row = []
            for c in range(8):
                wv = w_v[h, pl.ds(c * 16, 16)]
                row.append(wv * wv)
            w2.append(row)

        def start(block, lb, rb, nlb, nrb, sem):
            off = block * B
            idx_l = src_v.at[pl.ds(off, B)]
            idx_r = dst_v.at[pl.ds(off, B)]
            pltpu.async_copy(lf_hbm.at[idx_l], lb, sem)
            pltpu.async_copy(rf_hbm.at[idx_r], rb, sem)
            pltpu.async_copy(nl_hbm.at[idx_l], nlb, sem)
            pltpu.async_copy(nr_hbm.at[idx_r], nrb, sem)

        def wait(lb, rb, nlb, nrb, sem):
            idx = src_v.at[pl.ds(0, B)]
            pltpu.make_async_copy(lf_hbm.at[idx], lb, sem).wait()
            pltpu.make_async_copy(rf_hbm.at[idx], rb, sem).wait()
            pltpu.make_async_copy(nl_hbm.at[idx], nlb, sem).wait()
            pltpu.make_async_copy(nr_hbm.at[idx], nrb, sem).wait()

        lane = lax.iota(jnp.int32, 16)

        def compute(block, lb, rb, nlb, nrb):
            def group(g, carry):
                sims = jnp.zeros((16,), jnp.float32)
                for j in range(16):
                    e = g * 16 + j
                    lcs = [lb[e, pl.ds(c * 16, 16)] for c in range(8)]
                    rcs = [rb[e, pl.ds(c * 16, 16)] for c in range(8)]
                    qs = [lcs[c] * rcs[c] for c in range(8)]
                    lv = nlb[e, pl.ds(0, 16)]
                    rv = nrb[e, pl.ds(0, 16)]
                    tot = None
                    for h in range(H):
                        acc = qs[0] * w2[h][0]
                        for c in range(1, 8):
                            acc = acc + qs[c] * w2[h][c]
                        term = acc * (lv[h] * rv[h])
                        tot = term if tot is None else tot + term
                    sim = jnp.sum(tot) * jnp.float32(1.0 / H)
                    sims = jnp.where(lane == j, sim, sims)
                sims = jnp.where(sims < SIM_T, jnp.float32(0.0), sims)
                out_v[pl.ds(block * B + g * 16, 16)] = sims
                return carry
            lax.fori_loop(0, B // 16, group, 0)

        slots = tuple((lbuf.at[j], rbuf.at[j], nlbuf.at[j], nrbuf.at[j],
                       sems.at[j]) for j in range(2))
        start(0, *slots[0])
        start(1, *slots[1])

        def outer(kk, carry):
            for j in range(2):
                b = 2 * kk + j
                lb, rb, nlb, nrb, sem = slots[j]
                wait(lb, rb, nlb, nrb, sem)
                compute(b, lb, rb, nlb, nrb)

                @pl.when(b + 2 < NB)
                def _():
                    start(b + 2, lb, rb, nlb, nrb, sem)
            return carry

        lax.fori_loop(0, NB // 2, outer, 0)
        for j in range(NB - 2 * (NB // 2)):
            b = 2 * (NB // 2) + j
            lb, rb, nlb, nrb, sem = slots[j]
            wait(lb, rb, nlb, nrb, sem)
            compute(b, lb, rb, nlb, nrb)
        pltpu.sync_copy(out_v, out_hbm.at[pl.ds(base, EPT)])

    return run(lf, rf, nl, nr, edge_index, w2d)


def kernel(left_features, right_features, edge_index, W):
    w2d = W.reshape(H, D)
    nl, nr = _build_norms(left_features, right_features, w2d)
    return _edge_sim(left_features, right_features, nl, nr,
                     edge_index.reshape(2 * E), w2d)


# speedup vs baseline: 1.1976x; 1.0144x over previous
import functools

import jax
import jax.numpy as jnp
from jax import lax
from jax.experimental import pallas as pl
from jax.experimental.pallas import tpu as pltpu
from jax.experimental.pallas import tpu_sc as plsc

N = 10000
D = 128
E = 320000
H = 4
NR_ = 16
SIM_T = 0.1
EPS = 1e-8

NC, NS = 2, 16
NW = NC * NS
EPT = E // NW
B = 80
NB = EPT // B
BR = 1000


def _norms_body(l_ref, r_ref, w_ref, nl_ref, nr_ref):
    w = w_ref[:]
    for x_ref, o_ref in ((l_ref, nl_ref), (r_ref, nr_ref)):
        x = x_ref[:]
        cols = []
        for h in range(H):
            wl = x * w[h:h + 1, :]
            s = jnp.sum(wl * wl, axis=1, keepdims=True)
            cols.append(1.0 / jnp.maximum(jnp.sqrt(s), EPS))
        pad = jnp.zeros((BR, NR_ - H), jnp.float32)
        o_ref[:] = jnp.concatenate(cols + [pad], axis=1)


def _build_norms(left, right, w2d):
    return pl.pallas_call(
        _norms_body,
        grid=(N // BR,),
        in_specs=[
            pl.BlockSpec((BR, D), lambda i: (i, 0)),
            pl.BlockSpec((BR, D), lambda i: (i, 0)),
            pl.BlockSpec((H, D), lambda i: (0, 0)),
        ],
        out_specs=[
            pl.BlockSpec((BR, NR_), lambda i: (i, 0)),
            pl.BlockSpec((BR, NR_), lambda i: (i, 0)),
        ],
        out_shape=[
            jax.ShapeDtypeStruct((N, NR_), jnp.float32),
            jax.ShapeDtypeStruct((N, NR_), jnp.float32),
        ],
    )(left, right, w2d)


def _edge_sim(lf, rf, nl, nr, edge_index, w2d):
    mesh = plsc.VectorSubcoreMesh(core_axis_name="c", subcore_axis_name="s")

    @functools.partial(
        pl.kernel,
        out_type=jax.ShapeDtypeStruct((E,), jnp.float32),
        mesh=mesh,
        compiler_params=pltpu.CompilerParams(needs_layout_passes=False,
                                             use_tc_tiling_on_sc=False),
        scratch_types=[
            pltpu.VMEM((EPT,), jnp.int32),
            pltpu.VMEM((EPT,), jnp.int32),
            pltpu.VMEM((2, B, D), jnp.float32),
            pltpu.VMEM((2, B, D), jnp.float32),
            pltpu.VMEM((2, B, NR_), jnp.float32),
            pltpu.VMEM((2, B, NR_), jnp.float32),
            pltpu.VMEM((EPT,), jnp.float32),
            pltpu.VMEM((H, D), jnp.float32),
            pltpu.SemaphoreType.DMA((2,)),
        ],
    )
    def run(lf_hbm, rf_hbm, nl_hbm, nr_hbm, ei_hbm, w_hbm, out_hbm,
            src_v, dst_v, lbuf, rbuf, nlbuf, nrbuf, out_v, w_v, sems):
        wid = lax.axis_index("s") * NC + lax.axis_index("c")
        base = wid * EPT
        pltpu.sync_copy(ei_hbm.at[pl.ds(base, EPT)], src_v)
        pltpu.sync_copy(ei_hbm.at[pl.ds(E + base, EPT)], dst_v)
        pltpu.sync_copy(w_hbm, w_v)
        w2 = []
        for h in range(H):
            row = []
            for c in range(8):
                wv = w_v[h, pl.ds(c * 16, 16)]
                row.append(wv * wv)
            w2.append(row)

        def start(block, lb, rb, nlb, nrb, sem):
            off = block * B
            idx_l = src_v.at[pl.ds(off, B)]
            idx_r = dst_v.at[pl.ds(off, B)]
            pltpu.async_copy(lf_hbm.at[idx_l], lb, sem)
            pltpu.async_copy(rf_hbm.at[idx_r], rb, sem)
            pltpu.async_copy(nl_hbm.at[idx_l], nlb, sem)
            pltpu.async_copy(nr_hbm.at[idx_r], nrb, sem)

        def wait(lb, rb, nlb, nrb, sem):
            idx = src_v.at[pl.ds(0, B)]
            pltpu.make_async_copy(lf_hbm.at[idx], lb, sem).wait()
            pltpu.make_async_copy(rf_hbm.at[idx], rb, sem).wait()
            pltpu.make_async_copy(nl_hbm.at[idx], nlb, sem).wait()
            pltpu.make_async_copy(nr_hbm.at[idx], nrb, sem).wait()

        lane = lax.iota(jnp.int32, 16)

        def compute(block, lb, rb, nlb, nrb):
            def group(g, carry):
                sims = jnp.zeros((16,), jnp.float32)
                for j in range(16):
                    e = g * 16 + j
                    lv = nlb[e, pl.ds(0, 16)]
                    rv = nrb[e, pl.ds(0, 16)]
                    q = lb[e, pl.ds(0, 16)] * rb[e, pl.ds(0, 16)]
                    accs = [q * w2[h][0] for h in range(H)]
                    for c in range(1, 8):
                        q = lb[e, pl.ds(c * 16, 16)] * rb[e, pl.ds(c * 16, 16)]
                        for h in range(H):
                            accs[h] = accs[h] + q * w2[h][c]
                    tot = None
                    for h in range(H):
                        term = accs[h] * (lv[h] * rv[h])
                        tot = term if tot is None else tot + term
                    sim = jnp.sum(tot) * jnp.float32(1.0 / H)
                    sims = jnp.where(lane == j, sim, sims)
                sims = jnp.where(sims < SIM_T, jnp.float32(0.0), sims)
                out_v[pl.ds(block * B + g * 16, 16)] = sims
                return carry
            lax.fori_loop(0, B // 16, group, 0)

        slots = tuple((lbuf.at[j], rbuf.at[j], nlbuf.at[j], nrbuf.at[j],
                       sems.at[j]) for j in range(2))
        start(0, *slots[0])
        start(1, *slots[1])

        def outer(kk, carry):
            for j in range(2):
                b = 2 * kk + j
                lb, rb, nlb, nrb, sem = slots[j]
                wait(lb, rb, nlb, nrb, sem)
                compute(b, lb, rb, nlb, nrb)

                @pl.when(b + 2 < NB)
                def _():
                    start(b + 2, lb, rb, nlb, nrb, sem)
            return carry

        lax.fori_loop(0, NB // 2, outer, 0)
        for j in range(NB - 2 * (NB // 2)):
            b = 2 * (NB // 2) + j
            lb, rb, nlb, nrb, sem = slots[j]
            wait(lb, rb, nlb, nrb, sem)
            compute(b, lb, rb, nlb, nrb)
        pltpu.sync_copy(out_v, out_hbm.at[pl.ds(base, EPT)])

    return run(lf, rf, nl, nr, edge_index, w2d)


def kernel(left_features, right_features, edge_index, W):
    w2d = W.reshape(H, D)
    nl, nr = _build_norms(left_features, right_features, w2d)
    return _edge_sim(left_features, right_features, nl, nr,
                     edge_index.reshape(2 * E), w2d)


# speedup vs baseline: 1.4766x; 1.2330x over previous
import functools

import jax
import jax.numpy as jnp
from jax import lax
from jax.experimental import pallas as pl
from jax.experimental.pallas import tpu as pltpu
from jax.experimental.pallas import tpu_sc as plsc

N = 10000
D = 128
E = 320000
H = 4
NR_ = 16
SIM_T = 0.1
EPS = 1e-8

NC, NS = 2, 16
NW = NC * NS
EPT = E // NW
B = 80
NB = EPT // B
BR = 1000


def _norms_body(l_ref, r_ref, w_ref, nl_ref, nr_ref):
    w = w_ref[:]
    for x_ref, o_ref in ((l_ref, nl_ref), (r_ref, nr_ref)):
        x = x_ref[:]
        cols = []
        for h in range(H):
            wl = x * w[h:h + 1, :]
            s = jnp.sum(wl * wl, axis=1, keepdims=True)
            cols.append(1.0 / jnp.maximum(jnp.sqrt(s), EPS))
        pad = jnp.zeros((BR, NR_ - H), jnp.float32)
        o_ref[:] = jnp.concatenate(cols + [pad], axis=1)


def _build_norms(left, right, w2d):
    return pl.pallas_call(
        _norms_body,
        grid=(N // BR,),
        in_specs=[
            pl.BlockSpec((BR, D), lambda i: (i, 0)),
            pl.BlockSpec((BR, D), lambda i: (i, 0)),
            pl.BlockSpec((H, D), lambda i: (0, 0)),
        ],
        out_specs=[
            pl.BlockSpec((BR, NR_), lambda i: (i, 0)),
            pl.BlockSpec((BR, NR_), lambda i: (i, 0)),
        ],
        out_shape=[
            jax.ShapeDtypeStruct((N, NR_), jnp.float32),
            jax.ShapeDtypeStruct((N, NR_), jnp.float32),
        ],
    )(left, right, w2d)


def _edge_sim(lf, rf, nl, nr, edge_index, w2d):
    mesh = plsc.VectorSubcoreMesh(core_axis_name="c", subcore_axis_name="s")

    @functools.partial(
        pl.kernel,
        out_type=jax.ShapeDtypeStruct((E,), jnp.float32),
        mesh=mesh,
        compiler_params=pltpu.CompilerParams(needs_layout_passes=False,
                                             use_tc_tiling_on_sc=False),
        scratch_types=[
            pltpu.VMEM((EPT,), jnp.int32),
            pltpu.VMEM((EPT,), jnp.int32),
            pltpu.VMEM((2, B, D), jnp.float32),
            pltpu.VMEM((2, B, D), jnp.float32),
            pltpu.VMEM((2, B, NR_), jnp.float32),
            pltpu.VMEM((2, B, NR_), jnp.float32),
            pltpu.VMEM((EPT,), jnp.float32),
            pltpu.VMEM((H, D), jnp.float32),
            pltpu.SemaphoreType.DMA((2,)),
        ],
    )
    def run(lf_hbm, rf_hbm, nl_hbm, nr_hbm, ei_hbm, w_hbm, out_hbm,
            src_v, dst_v, lbuf, rbuf, nlbuf, nrbuf, out_v, w_v, sems):
        wid = lax.axis_index("s") * NC + lax.axis_index("c")
        base = wid * EPT
        pltpu.sync_copy(ei_hbm.at[pl.ds(base, EPT)], src_v)
        pltpu.sync_copy(ei_hbm.at[pl.ds(E + base, EPT)], dst_v)
        pltpu.sync_copy(w_hbm, w_v)
        w2 = []
        for h in range(H):
            row = []
            for c in range(8):
                wv = w_v[h, pl.ds(c * 16, 16)]
                row.append(wv * wv * jnp.float32(1.0 / H))
            w2.append(row)

        def start(block, lb, rb, nlb, nrb, sem):
            off = block * B
            idx_l = src_v.at[pl.ds(off, B)]
            idx_r = dst_v.at[pl.ds(off, B)]
            pltpu.async_copy(lf_hbm.at[idx_l], lb, sem)
            pltpu.async_copy(rf_hbm.at[idx_r], rb, sem)
            pltpu.async_copy(nl_hbm.at[idx_l], nlb, sem)
            pltpu.async_copy(nr_hbm.at[idx_r], nrb, sem)

        def wait(lb, rb, nlb, nrb, sem):
            idx = src_v.at[pl.ds(0, B)]
            pltpu.make_async_copy(lf_hbm.at[idx], lb, sem).wait()
            pltpu.make_async_copy(rf_hbm.at[idx], rb, sem).wait()
            pltpu.make_async_copy(nl_hbm.at[idx], nlb, sem).wait()
            pltpu.make_async_copy(nr_hbm.at[idx], nrb, sem).wait()

        lane = lax.iota(jnp.int32, 16)

        def compute(block, lb, rb, nlb, nrb):
            def group(g, carry):
                sims = jnp.zeros((16,), jnp.float32)
                for j in range(16):
                    e = g * 16 + j
                    lv = nlb[e, pl.ds(0, 16)]
                    rv = nrb[e, pl.ds(0, 16)]
                    q = lb[e, pl.ds(0, 16)] * rb[e, pl.ds(0, 16)]
                    accs = [q * w2[h][0] for h in range(H)]
                    for c in range(1, 8):
                        q = lb[e, pl.ds(c * 16, 16)] * rb[e, pl.ds(c * 16, 16)]
                        for h in range(H):
                            accs[h] = accs[h] + q * w2[h][c]
                    tot = None
                    for h in range(H):
                        term = accs[h] * (lv[h] * rv[h])
                        tot = term if tot is None else tot + term
                    sim = jnp.sum(tot)
                    sims = jnp.where(lane == j, sim, sims)
                sims = jnp.where(sims < SIM_T, jnp.float32(0.0), sims)
                out_v[pl.ds(block * B + g * 16, 16)] = sims
                return carry
            lax.fori_loop(0, B // 16, group, 0)

        slots = tuple((lbuf.at[j], rbuf.at[j], nlbuf.at[j], nrbuf.at[j],
                       sems.at[j]) for j in range(2))
        start(0, *slots[0])
        start(1, *slots[1])

        def outer(kk, carry):
            for j in range(2):
                b = 2 * kk + j
                lb, rb, nlb, nrb, sem = slots[j]
                wait(lb, rb, nlb, nrb, sem)
                compute(b, lb, rb, nlb, nrb)

                @pl.when(b + 2 < NB)
                def _():
                    start(b + 2, lb, rb, nlb, nrb, sem)
            return carry

        lax.fori_loop(0, NB // 2, outer, 0)
        for j in range(NB - 2 * (NB // 2)):
            b = 2 * (NB // 2) + j
            lb, rb, nlb, nrb, sem = slots[j]
            wait(lb, rb, nlb, nrb, sem)
            compute(b, lb, rb, nlb, nrb)
        pltpu.sync_copy(out_v, out_hbm.at[pl.ds(base, EPT)])

    return run(lf, rf, nl, nr, edge_index, w2d)


def kernel(left_features, right_features, edge_index, W):
    w2d = W.reshape(H, D)
    nl, nr = _build_norms(left_features, right_features, w2d)
    return _edge_sim(left_features, right_features, nl, nr,
                     edge_index.reshape(2 * E), w2d)
